# Initial kernel scaffold; baseline (speedup 1.0000x reference)
import jax
import jax.numpy as jnp
from jax.experimental import pallas as pl


def kernel(states, adj_list, adj_list_mask, lut, W, b):
    raise NotImplementedError("write your pallas kernel here")


# speedup vs baseline: 7.3498x; 7.3498x over previous
import functools

import jax
import jax.numpy as jnp
from jax import lax
from jax.experimental import pallas as pl
from jax.experimental.pallas import tpu as pltpu
from jax.experimental.pallas import tpu_sc as plsc

N_NODES_C = 100000
N_IN_C = 1024
K_C = 8
N_OUT_C = 128
BATCH_C = 32

N_RES = N_NODES_C - N_IN_C
NODE_BLK = 512
N_GRID = (N_RES + NODE_BLK - 1) // NODE_BLK
N_PAD = N_GRID * NODE_BLK
IDX_TOTAL = K_C * N_PAD
NUM_TILES = 32
IDX_PER_TILE = IDX_TOTAL // NUM_TILES
IDX_CHUNK = 6208
N_CHUNKS = IDX_PER_TILE // IDX_CHUNK


def _pack_states_body(s_ref, o_ref):
    s = s_ref[...]
    shifts = lax.broadcasted_iota(jnp.int32, (BATCH_C, 1), 0)
    o_ref[...] = jnp.sum(jnp.left_shift(s, shifts), axis=0, keepdims=True)


def _pack_states(states):
    blk = 4096
    grid = (N_NODES_C + blk - 1) // blk
    return pl.pallas_call(
        _pack_states_body,
        grid=(grid,),
        in_specs=[pl.BlockSpec((BATCH_C, blk), lambda i: (0, i))],
        out_specs=pl.BlockSpec((1, blk), lambda i: (0, i)),
        out_shape=jax.ShapeDtypeStruct((1, N_NODES_C), jnp.int32),
    )(states)


def _pack_lut_body(l_ref, o_ref):
    lut_blk = l_ref[...]
    jrow = lax.broadcasted_iota(jnp.int32, (1, 32), 1)
    for w in range(8):
        sl = lut_blk[:, w * 32:(w + 1) * 32]
        word = jnp.sum(jnp.left_shift(sl, jrow), axis=1, keepdims=True)
        o_ref[:, w:w + 1] = word


def _pack_lut(lut):
    blk = 2048
    grid = (N_NODES_C + blk - 1) // blk
    return pl.pallas_call(
        _pack_lut_body,
        grid=(grid,),
        in_specs=[pl.BlockSpec((blk, 256), lambda i: (i, 0))],
        out_specs=pl.BlockSpec((blk, 8), lambda i: (i, 0)),
        out_shape=jax.ShapeDtypeStruct((N_NODES_C, 8), jnp.int32),
    )(lut)


def _sc_gather(ps_flat, idx_flat):
    mesh = plsc.VectorSubcoreMesh(core_axis_name="c", subcore_axis_name="s")

    @functools.partial(
        pl.kernel,
        mesh=mesh,
        out_type=jax.ShapeDtypeStruct((IDX_TOTAL,), jnp.int32),
        scratch_types=[
            pltpu.VMEM((N_NODES_C,), jnp.int32),
            pltpu.VMEM((IDX_CHUNK,), jnp.int32),
            pltpu.VMEM((IDX_CHUNK,), jnp.int32),
        ],
        compiler_params=pltpu.CompilerParams(needs_layout_passes=False),
    )
    def k(ps_hbm, idx_hbm, out_hbm, table_v, idx_v, out_v):
        wid = lax.axis_index("s") * 2 + lax.axis_index("c")
        base = wid * IDX_PER_TILE
        pltpu.sync_copy(ps_hbm, table_v)

        def chunk_body(ci, _):
            off = base + ci * IDX_CHUNK
            pltpu.sync_copy(idx_hbm.at[pl.ds(off, IDX_CHUNK)], idx_v)

            def gather_body(t, __):
                iv = idx_v[pl.ds(t * 16, 16)]
                out_v[pl.ds(t * 16, 16)] = plsc.load_gather(table_v, [iv])
                return __

            lax.fori_loop(0, IDX_CHUNK // 16, gather_body, 0)
            pltpu.sync_copy(out_v, out_hbm.at[pl.ds(off, IDX_CHUNK)])
            return _

        lax.fori_loop(0, N_CHUNKS, chunk_body, 0)

    return k(ps_flat, idx_flat)


def _update_readout_body(g_ref, m_ref, lp_ref, ps_ref, w_ref, b_ref, o_ref):
    i = pl.program_id(0)
    nvalid = N_RES - i * NODE_BLK
    lane = lax.broadcasted_iota(jnp.int32, (1, NODE_BLK), 1)
    valid = lane < nvalid

    k_iota = lax.broadcasted_iota(jnp.int32, (K_C, 1), 0)
    pow2 = jnp.int32(128) >> k_iota
    w_iota = lax.broadcasted_iota(jnp.int32, (8, 1), 0)

    g = g_ref[...] * m_ref[...]
    lp = lp_ref[...]
    ps = ps_ref[...]
    nb0 = jnp.sum(m_ref[...], axis=0, keepdims=True) == 0

    rows = []
    for b in range(BATCH_C):
        bits = lax.shift_right_logical(g, b) & 1
        idx = jnp.sum(bits * pow2, axis=0, keepdims=True)
        wsel = idx >> 5
        eq = (jnp.broadcast_to(wsel, (8, NODE_BLK)) == w_iota).astype(jnp.int32)
        lutw = jnp.sum(lp * eq, axis=0, keepdims=True)
        bit = lax.shift_right_logical(lutw, idx & 31) & 1
        old = lax.shift_right_logical(ps, b) & 1
        rows.append(jnp.where(nb0, old, bit))
    res = jnp.concatenate(rows, axis=0).astype(jnp.float32)

    sub = lax.broadcasted_iota(jnp.int32, (NODE_BLK, 1), 0)
    wblk = jnp.where(sub < nvalid, w_ref[...], 0.0)
    res = jnp.where(valid, res, 0.0)

    @pl.when(i == 0)
    def _init():
        o_ref[...] = jnp.zeros_like(o_ref)

    o_ref[...] += jnp.dot(res, wblk, preferred_element_type=jnp.float32,
                          precision=lax.Precision.HIGHEST)

    @pl.when(i == N_GRID - 1)
    def _fin():
        o_ref[...] = jax.nn.sigmoid(o_ref[...] + b_ref[...])


def _update_readout(g, mask_t, lutp_t, ps_r, W, bias):
    return pl.pallas_call(
        _update_readout_body,
        grid=(N_GRID,),
        in_specs=[
            pl.BlockSpec((K_C, NODE_BLK), lambda i: (0, i)),
            pl.BlockSpec((K_C, NODE_BLK), lambda i: (0, i)),
            pl.BlockSpec((8, NODE_BLK), lambda i: (0, i)),
            pl.BlockSpec((1, NODE_BLK), lambda i: (0, i)),
            pl.BlockSpec((NODE_BLK, N_OUT_C), lambda i: (i, 0)),
            pl.BlockSpec((1, N_OUT_C), lambda i: (0, 0)),
        ],
        out_specs=pl.BlockSpec((BATCH_C, N_OUT_C), lambda i: (0, 0)),
        out_shape=jax.ShapeDtypeStruct((BATCH_C, N_OUT_C), jnp.float32),
    )(g, mask_t, lutp_t, ps_r, W, bias)


def kernel(states, adj_list, adj_list_mask, lut, W, b):
    ps2 = _pack_states(states)
    lutp = _pack_lut(lut)

    pad = N_PAD - N_RES
    adj_t = jnp.pad(adj_list[N_IN_C:].T, ((0, 0), (0, pad)))
    mask_t = jnp.pad(adj_list_mask[N_IN_C:].T, ((0, 0), (0, pad)))
    lutp_t = jnp.pad(lutp[N_IN_C:].T, ((0, 0), (0, pad)))
    ps_r = jnp.pad(ps2[:, N_IN_C:], ((0, 0), (0, pad)))

    g_flat = _sc_gather(ps2.reshape(N_NODES_C), adj_t.reshape(IDX_TOTAL))
    g = g_flat.reshape(K_C, N_PAD)

    return _update_readout(g, mask_t, lutp_t, ps_r, W,
                           b.reshape(1, N_OUT_C))
